# Initial kernel scaffold; baseline (speedup 1.0000x reference)
import jax
import jax.numpy as jnp
from jax.experimental import pallas as pl


def kernel(x, weight):
    raise NotImplementedError("write your pallas kernel here")


# speedup vs baseline: 1.5787x; 1.5787x over previous
import jax
import jax.numpy as jnp
from jax import lax
from jax.experimental import pallas as pl


_N_EMB = 1024


def _vq_body(x_ref, w_ref, res_ref, idx_ref):
    xb = x_ref[0]
    w = w_ref[...]
    P = xb.shape[1]
    scores = lax.dot_general(
        xb, w, (((0,), (0,)), ((), ())),
        preferred_element_type=jnp.float32,
        precision=lax.Precision.HIGHEST,
    )
    e2 = jnp.sum(w * w, axis=0, keepdims=True)
    dist = e2 - 2.0 * scores
    m = jnp.min(dist, axis=1, keepdims=True)
    ids = lax.broadcasted_iota(jnp.int32, (P, _N_EMB), 1)
    idx = jnp.min(jnp.where(dist == m, ids, jnp.int32(_N_EMB)), axis=1)
    idx_ref[0, 0, :] = idx
    onehot = (ids == idx[:, None]).astype(jnp.float32)
    res_ref[0] = lax.dot_general(
        w, onehot, (((1,), (1,)), ((), ())),
        preferred_element_type=jnp.float32,
        precision=lax.Precision.HIGHEST,
    )


def kernel(x, weight):
    B, D, H, W = x.shape
    P = H * W
    x3 = x.reshape(B, D, P)
    res, idx = pl.pallas_call(
        _vq_body,
        grid=(B,),
        in_specs=[
            pl.BlockSpec((1, D, P), lambda b: (b, 0, 0)),
            pl.BlockSpec((D, _N_EMB), lambda b: (0, 0)),
        ],
        out_specs=[
            pl.BlockSpec((1, D, P), lambda b: (b, 0, 0)),
            pl.BlockSpec((1, 1, P), lambda b: (b, 0, 0)),
        ],
        out_shape=[
            jax.ShapeDtypeStruct((B, D, P), jnp.float32),
            jax.ShapeDtypeStruct((B, 1, P), jnp.int32),
        ],
    )(x3, weight)
    return res.reshape(B, D, H, W), idx.reshape(B, H, W)


# speedup vs baseline: 2.7203x; 1.7232x over previous
import jax
import jax.numpy as jnp
from jax import lax
from jax.experimental import pallas as pl


_N_EMB = 1024


def _vq_body(xt_ref, w_ref, res_ref, idx_ref):
    xt = xt_ref[...]
    w = w_ref[...]
    M = xt.shape[0]
    scores = lax.dot_general(
        xt, w, (((1,), (0,)), ((), ())),
        preferred_element_type=jnp.float32,
        precision=lax.Precision.HIGHEST,
    )
    e2 = jnp.sum(w * w, axis=0, keepdims=True)
    dist = e2 - 2.0 * scores
    m = jnp.min(dist, axis=1, keepdims=True)
    ids = lax.broadcasted_iota(jnp.int32, (M, _N_EMB), 1)
    idx = jnp.min(jnp.where(dist == m, ids, jnp.int32(_N_EMB)),
                  axis=1, keepdims=True)
    idx_ref[...] = idx
    onehot = (ids == idx).astype(jnp.float32)
    B = res_ref.shape[0]
    P = M // B
    for b in range(B):
        res_ref[b] = lax.dot_general(
            w, onehot[b * P:(b + 1) * P], (((1,), (1,)), ((), ())),
            preferred_element_type=jnp.float32,
            precision=lax.Precision.HIGHEST,
        )


def kernel(x, weight):
    B, D, H, W = x.shape
    P = H * W
    M = B * P
    xt = x.reshape(B, D, P).transpose(0, 2, 1).reshape(M, D)
    res, idx = pl.pallas_call(
        _vq_body,
        out_shape=[
            jax.ShapeDtypeStruct((B, D, P), jnp.float32),
            jax.ShapeDtypeStruct((M, 1), jnp.int32),
        ],
    )(xt, weight)
    return res.reshape(B, D, H, W), idx.reshape(B, H, W)
